# Initial kernel scaffold; baseline (speedup 1.0000x reference)
import jax
import jax.numpy as jnp
from jax.experimental import pallas as pl


def kernel(out, target, neighbor_idxs, neighbor_weights):
    raise NotImplementedError("write your pallas kernel here")


# speedup vs baseline: 18.0325x; 18.0325x over previous
import functools

import jax
import jax.numpy as jnp
from jax import lax
from jax.experimental import pallas as pl
from jax.experimental.pallas import tpu as pltpu
from jax.experimental.pallas import tpu_sc as plsc

N = 100000
K = 10
B = 4
C = 3
P = B * C
CH = 400
NCH = N // CH
JPC = CH // 16
HALF = NCH // 2

_info = plsc.get_sparse_core_info()
_NC = _info.num_cores
_NS = _info.num_subcores


def _sub_body(a_ref, b_ref, o_ref):
    o_ref[...] = a_ref[...] - b_ref[...]


def _diff(out, target):
    a = out.reshape(1200, 1000)
    b = target.reshape(1200, 1000)
    d = pl.pallas_call(
        _sub_body,
        out_shape=jax.ShapeDtypeStruct((1200, 1000), jnp.float32),
        grid=(10,),
        in_specs=[
            pl.BlockSpec((120, 1000), lambda i: (i, 0)),
            pl.BlockSpec((120, 1000), lambda i: (i, 0)),
        ],
        out_specs=pl.BlockSpec((120, 1000), lambda i: (i, 0)),
    )(a, b)
    return d.reshape(B, N, C)


def _sc_body(dp, idxc, wc, lossp, plane, idxb, wb, outb):
    wid = lax.axis_index("s") * _NC + lax.axis_index("c")

    @pl.when(wid < P * 2)
    def _():
        p = wid // 2
        h = wid % 2
        pltpu.sync_copy(dp.at[pl.ds(p * N, N)], plane)
        c0 = h * HALF

        def chunk(cb_rel, carry):
            cb = c0 + cb_rel
            pltpu.sync_copy(idxc.at[pl.ds(cb * (K * CH), K * CH)], idxb)
            pltpu.sync_copy(wc.at[pl.ds(cb * (K * CH), K * CH)], wb)

            def grp(j, inner):
                r0 = j * 16
                acc = jnp.zeros((16,), jnp.float32)
                for k in range(K):
                    ii = idxb[pl.ds(k * CH + r0, 16)]
                    g = plsc.load_gather(plane, [ii])
                    ww = wb[pl.ds(k * CH + r0, 16)]
                    acc = acc + g * ww
                ctr = plane[pl.ds(cb * CH + r0, 16)]
                v = ctr + acc
                outb[pl.ds(r0, 16)] = v * v
                return inner

            lax.fori_loop(0, JPC, grp, 0)
            pltpu.sync_copy(outb, lossp.at[pl.ds(p * N + cb * CH, CH)])
            return carry

        lax.fori_loop(0, HALF, chunk, 0)


_sc_kernel = functools.partial(
    pl.kernel,
    mesh=plsc.VectorSubcoreMesh(core_axis_name="c", subcore_axis_name="s"),
    compiler_params=pltpu.CompilerParams(needs_layout_passes=False),
    out_type=jax.ShapeDtypeStruct((P * N,), jnp.float32),
    scratch_types=[
        pltpu.VMEM((N,), jnp.float32),
        pltpu.VMEM((K * CH,), jnp.int32),
        pltpu.VMEM((K * CH,), jnp.float32),
        pltpu.VMEM((CH,), jnp.float32),
    ],
)(_sc_body)


def kernel(out, target, neighbor_idxs, neighbor_weights):
    idx32 = neighbor_idxs.astype(jnp.int32)
    idxc = idx32.T.reshape(K, NCH, CH).transpose(1, 0, 2).reshape(-1)
    wc = neighbor_weights.T.reshape(K, NCH, CH).transpose(1, 0, 2).reshape(-1)
    d = _diff(out, target)
    dp = d.transpose(0, 2, 1).reshape(P * N)
    lossp = _sc_kernel(dp, idxc, wc)
    return lossp.reshape(B, C, N).transpose(0, 2, 1)


# speedup vs baseline: 19.3519x; 1.0732x over previous
import functools

import jax
import jax.numpy as jnp
from jax import lax
from jax.experimental import pallas as pl
from jax.experimental.pallas import tpu as pltpu
from jax.experimental.pallas import tpu_sc as plsc

N = 100000
K = 10
B = 4
C = 3
P = B * C
CH = 400
NCH = N // CH
JPC = CH // 16
HALF = NCH // 2
CW = K * CH

_info = plsc.get_sparse_core_info()
_NC = _info.num_cores
_NS = _info.num_subcores


def _sub_body(a_ref, b_ref, o_ref):
    o_ref[...] = a_ref[...] - b_ref[...]


def _diff(out, target):
    a = out.reshape(1200, 1000)
    b = target.reshape(1200, 1000)
    d = pl.pallas_call(
        _sub_body,
        out_shape=jax.ShapeDtypeStruct((1200, 1000), jnp.float32),
        grid=(10,),
        in_specs=[
            pl.BlockSpec((120, 1000), lambda i: (i, 0)),
            pl.BlockSpec((120, 1000), lambda i: (i, 0)),
        ],
        out_specs=pl.BlockSpec((120, 1000), lambda i: (i, 0)),
    )(a, b)
    return d.reshape(B, N, C)


def _sc_body(dp, idxf, wf, lossp, plane,
             i0, i1, w0, w1, o0, o1,
             si0, si1, sw0, sw1, so0, so1):
    wid = lax.axis_index("s") * _NC + lax.axis_index("c")
    ibufs, wbufs, obufs = (i0, i1), (w0, w1), (o0, o1)
    isems, wsems, osems = (si0, si1), (sw0, sw1), (so0, so1)

    @pl.when(wid < P * 2)
    def _():
        p = wid // 2
        h = wid % 2
        pltpu.sync_copy(dp.at[pl.ds(p * N, N)], plane)
        c0 = h * HALF
        vK = lax.iota(jnp.int32, 16) * K

        def start_in(cb, par):
            pltpu.async_copy(idxf.at[pl.ds(cb * CW, CW)], ibufs[par], isems[par])
            pltpu.async_copy(wf.at[pl.ds(cb * CW, CW)], wbufs[par], wsems[par])

        start_in(c0, 0)

        def do_chunk(cb2, par):
                cb = c0 + cb2 * 2 + par

                @pl.when(cb + 1 < c0 + HALF)
                def _prefetch():
                    start_in(cb + 1, 1 - par)

                pltpu.make_async_copy(
                    idxf.at[pl.ds(cb * CW, CW)], ibufs[par], isems[par]).wait()
                pltpu.make_async_copy(
                    wf.at[pl.ds(cb * CW, CW)], wbufs[par], wsems[par]).wait()

                @pl.when(cb2 > 0)
                def _reclaim():
                    pltpu.make_async_copy(
                        obufs[par], lossp.at[pl.ds(0, CH)], osems[par]).wait()

                def grp(j, inner):
                    r0 = j * 16
                    base = j * (16 * K)
                    acc = jnp.zeros((16,), jnp.float32)
                    for k in range(K):
                        sel = vK + (base + k)
                        ii = plsc.load_gather(ibufs[par], [sel])
                        g = plsc.load_gather(plane, [ii])
                        ww = plsc.load_gather(wbufs[par], [sel])
                        acc = acc + g * ww
                    ctr = plane[pl.ds(cb * CH + r0, 16)]
                    v = ctr + acc
                    obufs[par][pl.ds(r0, 16)] = v * v
                    return inner

                lax.fori_loop(0, JPC, grp, 0)
                pltpu.async_copy(
                    obufs[par], lossp.at[pl.ds(p * N + cb * CH, CH)], osems[par])

        def chunk2(cb2, carry):
            for par in range(2):
                do_chunk(cb2, par)
            return carry

        lax.fori_loop(0, HALF // 2, chunk2, 0)
        if HALF % 2:
            do_chunk(HALF // 2, 0)
        for par in range(2):
            pltpu.make_async_copy(
                obufs[par], lossp.at[pl.ds(0, CH)], osems[par]).wait()


_sc_kernel = functools.partial(
    pl.kernel,
    mesh=plsc.VectorSubcoreMesh(core_axis_name="c", subcore_axis_name="s"),
    compiler_params=pltpu.CompilerParams(needs_layout_passes=False),
    out_type=jax.ShapeDtypeStruct((P * N,), jnp.float32),
    scratch_types=[
        pltpu.VMEM((N,), jnp.float32),
        pltpu.VMEM((CW,), jnp.int32),
        pltpu.VMEM((CW,), jnp.int32),
        pltpu.VMEM((CW,), jnp.float32),
        pltpu.VMEM((CW,), jnp.float32),
        pltpu.VMEM((CH,), jnp.float32),
        pltpu.VMEM((CH,), jnp.float32),
        pltpu.SemaphoreType.DMA,
        pltpu.SemaphoreType.DMA,
        pltpu.SemaphoreType.DMA,
        pltpu.SemaphoreType.DMA,
        pltpu.SemaphoreType.DMA,
        pltpu.SemaphoreType.DMA,
    ],
)(_sc_body)


def kernel(out, target, neighbor_idxs, neighbor_weights):
    idxf = neighbor_idxs.astype(jnp.int32).reshape(-1)
    wf = neighbor_weights.reshape(-1)
    d = _diff(out, target)
    dp = d.transpose(0, 2, 1).reshape(P * N)
    lossp = _sc_kernel(dp, idxf, wf)
    return lossp.reshape(B, C, N).transpose(0, 2, 1)
